# baseline (speedup 1.0000x reference)
import functools

import jax
import jax.numpy as jnp
from jax.experimental import pallas as pl

F = 16
_WIDTHS = [F // (j + 1) for j in range(F)]


def _top16(mat, width):
    iota = jax.lax.broadcasted_iota(jnp.int32, mat.shape, 1)
    neg_inf = jnp.float32(-jnp.inf)
    vals, idxs = [], []
    run = mat
    for _ in range(F):
        m = jnp.max(run, axis=1, keepdims=True)
        pos = jnp.argmax(run, axis=1).astype(jnp.int32)[:, None]
        vals.append(m)
        idxs.append(pos)
        run = jnp.where(iota == pos, neg_inf, run)
    return jnp.concatenate(vals, axis=1), jnp.concatenate(idxs, axis=1)


def _sim_topk_kernel(x_ref, wq_ref, keys_ref, vals_ref, idx_ref, *, num_keys):
    ph = pl.program_id(1)
    q = jax.lax.dot_general(x_ref[...], wq_ref[ph], (((1,), (0,)), ((), ())))
    sim = jax.lax.dot_general(q, keys_ref[ph], (((1,), (1,)), ((), ())))
    v, i = _top16(sim, num_keys)
    vals_ref[0] = v
    idx_ref[0] = i


def _combine_kernel(vals_ref, idx_ref, score_ref, out_ref, *, heads, num_keys):
    for h in range(heads):
        a, bv = vals_ref[h], vals_ref[heads + h]
        ai = idx_ref[h]
        bi = idx_ref[heads + h] * num_keys
        cand = jnp.concatenate(
            [a[:, :w] + bv[:, j:j + 1] for j, w in enumerate(_WIDTHS)], axis=1)
        candi = jnp.concatenate(
            [ai[:, :w] + bi[:, j:j + 1] for j, w in enumerate(_WIDTHS)], axis=1)
        width = cand.shape[1]
        iota = jax.lax.broadcasted_iota(jnp.int32, cand.shape, 1)
        neg_inf = jnp.float32(-jnp.inf)
        for t in range(F):
            m = jnp.max(cand, axis=1, keepdims=True)
            pos = jnp.min(jnp.where(cand == m, iota, width), axis=1,
                          keepdims=True)
            picked = jnp.max(jnp.where(iota == pos, candi, -1), axis=1)
            score_ref[:, h, t] = m[:, 0]
            out_ref[:, h, t] = picked
            cand = jnp.where(iota == pos, neg_inf, cand)


def kernel(x, Wq, keys):
    b, n, dim = x.shape
    p, num_keys, heads, dim_key = keys.shape
    ntok = b * n
    ph_total = p * heads
    xf = x.reshape(ntok, dim)
    wq3 = Wq.reshape(dim, ph_total, dim_key).transpose(1, 0, 2)
    kt = keys.transpose(0, 2, 1, 3).reshape(ph_total, num_keys, dim_key)

    t_tile = 1024
    while ntok % t_tile:
        t_tile //= 2

    body_a = functools.partial(_sim_topk_kernel, num_keys=num_keys)
    p1_vals, p1_idx = pl.pallas_call(
        body_a,
        grid=(ntok // t_tile, ph_total),
        in_specs=[
            pl.BlockSpec((t_tile, dim), lambda i, j: (i, 0)),
            pl.BlockSpec((ph_total, dim, dim_key), lambda i, j: (0, 0, 0)),
            pl.BlockSpec((ph_total, num_keys, dim_key), lambda i, j: (0, 0, 0)),
        ],
        out_specs=[
            pl.BlockSpec((1, t_tile, F), lambda i, j: (j, i, 0)),
            pl.BlockSpec((1, t_tile, F), lambda i, j: (j, i, 0)),
        ],
        out_shape=[
            jax.ShapeDtypeStruct((ph_total, ntok, F), jnp.float32),
            jax.ShapeDtypeStruct((ph_total, ntok, F), jnp.int32),
        ],
    )(xf, wq3, kt)

    t2 = 512
    while ntok % t2:
        t2 //= 2
    body_b = functools.partial(_combine_kernel, heads=heads, num_keys=num_keys)
    scores, idxs = pl.pallas_call(
        body_b,
        grid=(ntok // t2,),
        in_specs=[
            pl.BlockSpec((ph_total, t2, F), lambda i: (0, i, 0)),
            pl.BlockSpec((ph_total, t2, F), lambda i: (0, i, 0)),
        ],
        out_specs=[
            pl.BlockSpec((t2, heads, F), lambda i: (i, 0, 0)),
            pl.BlockSpec((t2, heads, F), lambda i: (i, 0, 0)),
        ],
        out_shape=[
            jax.ShapeDtypeStruct((ntok, heads, F), jnp.float32),
            jax.ShapeDtypeStruct((ntok, heads, F), jnp.int32),
        ],
    )(p1_vals, p1_idx)
    return scores.reshape(b, n, heads, F), idxs.reshape(b, n, heads, F)


# speedup vs baseline: 1.2658x; 1.2658x over previous
import functools

import jax
import jax.numpy as jnp
from jax.experimental import pallas as pl

F = 16
_WIDTHS = [F // (j + 1) for j in range(F)]


def _top16(mat, width):
    iota = jax.lax.broadcasted_iota(jnp.int32, mat.shape, 1)
    neg_inf = jnp.float32(-jnp.inf)
    vals, idxs = [], []
    run = mat
    for _ in range(F):
        m = jnp.max(run, axis=1, keepdims=True)
        pos = jnp.min(jnp.where(run == m, iota, width), axis=1, keepdims=True)
        vals.append(m)
        idxs.append(pos)
        run = jnp.where(iota == pos, neg_inf, run)
    return jnp.concatenate(vals, axis=1), jnp.concatenate(idxs, axis=1)


def _sim_topk_kernel(x_ref, wq_ref, keys_ref, vals_ref, idx_ref, *, num_keys):
    ph = pl.program_id(1)
    q = jax.lax.dot_general(x_ref[...], wq_ref[ph], (((1,), (0,)), ((), ())))
    sim = jax.lax.dot_general(q, keys_ref[ph], (((1,), (1,)), ((), ())))
    v, i = _top16(sim, num_keys)
    vals_ref[0] = v
    idx_ref[0] = i


def _combine_kernel(vals_ref, idx_ref, score_ref, out_ref, *, heads, num_keys):
    for h in range(heads):
        a, bv = vals_ref[h], vals_ref[heads + h]
        ai = idx_ref[h]
        bi = idx_ref[heads + h] * num_keys
        cand = jnp.concatenate(
            [a[:, :w] + bv[:, j:j + 1] for j, w in enumerate(_WIDTHS)], axis=1)
        candi = jnp.concatenate(
            [ai[:, :w] + bi[:, j:j + 1] for j, w in enumerate(_WIDTHS)], axis=1)
        width = cand.shape[1]
        iota = jax.lax.broadcasted_iota(jnp.int32, cand.shape, 1)
        neg_inf = jnp.float32(-jnp.inf)
        for t in range(F):
            m = jnp.max(cand, axis=1, keepdims=True)
            pos = jnp.min(jnp.where(cand == m, iota, width), axis=1,
                          keepdims=True)
            picked = jnp.max(jnp.where(iota == pos, candi, -1), axis=1)
            score_ref[:, h, t] = m[:, 0]
            out_ref[:, h, t] = picked
            cand = jnp.where(iota == pos, neg_inf, cand)


def kernel(x, Wq, keys):
    b, n, dim = x.shape
    p, num_keys, heads, dim_key = keys.shape
    ntok = b * n
    ph_total = p * heads
    xf = x.reshape(ntok, dim)
    wq3 = Wq.reshape(dim, ph_total, dim_key).transpose(1, 0, 2)
    kt = keys.transpose(0, 2, 1, 3).reshape(ph_total, num_keys, dim_key)

    t_tile = 1024
    while ntok % t_tile:
        t_tile //= 2

    body_a = functools.partial(_sim_topk_kernel, num_keys=num_keys)
    p1_vals, p1_idx = pl.pallas_call(
        body_a,
        grid=(ntok // t_tile, ph_total),
        in_specs=[
            pl.BlockSpec((t_tile, dim), lambda i, j: (i, 0)),
            pl.BlockSpec((ph_total, dim, dim_key), lambda i, j: (0, 0, 0)),
            pl.BlockSpec((ph_total, num_keys, dim_key), lambda i, j: (0, 0, 0)),
        ],
        out_specs=[
            pl.BlockSpec((1, t_tile, F), lambda i, j: (j, i, 0)),
            pl.BlockSpec((1, t_tile, F), lambda i, j: (j, i, 0)),
        ],
        out_shape=[
            jax.ShapeDtypeStruct((ph_total, ntok, F), jnp.float32),
            jax.ShapeDtypeStruct((ph_total, ntok, F), jnp.int32),
        ],
    )(xf, wq3, kt)

    t2 = 512
    while ntok % t2:
        t2 //= 2
    body_b = functools.partial(_combine_kernel, heads=heads, num_keys=num_keys)
    scores, idxs = pl.pallas_call(
        body_b,
        grid=(ntok // t2,),
        in_specs=[
            pl.BlockSpec((ph_total, t2, F), lambda i: (0, i, 0)),
            pl.BlockSpec((ph_total, t2, F), lambda i: (0, i, 0)),
        ],
        out_specs=[
            pl.BlockSpec((t2, heads, F), lambda i: (i, 0, 0)),
            pl.BlockSpec((t2, heads, F), lambda i: (i, 0, 0)),
        ],
        out_shape=[
            jax.ShapeDtypeStruct((ntok, heads, F), jnp.float32),
            jax.ShapeDtypeStruct((ntok, heads, F), jnp.int32),
        ],
    )(p1_vals, p1_idx)
    return scores.reshape(b, n, heads, F), idxs.reshape(b, n, heads, F)
